# baseline (device time: 167514 ns/iter reference)
import jax
import jax.numpy as jnp
from jax import lax
from jax.experimental import pallas as pl
from jax.experimental.pallas import tpu as pltpu

N_DEV = 4
N_TILE = 512


def kernel(x, w_mat, scale_x, scale_w):
    m_glob, k_loc = x.shape
    k_glob, n = w_mat.shape
    m_loc = m_glob // N_DEV
    n_steps = n // N_TILE

    def body(x_hbm, w_ref, sx_ref, sw_ref, out_ref,
             xf32, xsend, xg, recv_buf, copy_sem, send_sems, recv_sems):
        j = pl.program_id(0)
        k = pl.program_id(1)
        my = lax.axis_index("i")

        def mk_rdma(d):
            peer = lax.rem(my + d, N_DEV)
            return pltpu.make_async_remote_copy(
                src_ref=xsend.at[pl.ds(peer * m_loc, m_loc), :],
                dst_ref=recv_buf.at[d - 1],
                send_sem=send_sems.at[d - 1],
                recv_sem=recv_sems.at[d - 1],
                device_id=(peer,),
                device_id_type=pl.DeviceIdType.MESH,
            )

        @pl.when(jnp.logical_and(j == 0, k == 0))
        def _():
            cp = pltpu.make_async_copy(x_hbm, xf32, copy_sem)
            cp.start()

            barrier_sem = pltpu.get_barrier_semaphore()
            for d in range(1, N_DEV):
                peer = lax.rem(my + d, N_DEV)
                pl.semaphore_signal(
                    barrier_sem, inc=1,
                    device_id=(peer,), device_id_type=pl.DeviceIdType.MESH,
                )
            pl.semaphore_wait(barrier_sem, N_DEV - 1)
            cp.wait()

            xsend[:, :] = xf32[:, :].astype(jnp.float8_e4m3fn)
            for d in range(1, N_DEV):
                mk_rdma(d).start()

            xg[:, pl.ds(my * k_loc, k_loc)] = (
                xf32[pl.ds(my * m_loc, m_loc), :].astype(jnp.float8_e4m3fn)
            )

        for d in range(1, N_DEV):
            @pl.when(jnp.logical_and(j == 0, k == d))
            def _(d=d):
                rdma = mk_rdma(d)
                rdma.wait_send()
                rdma.wait_recv()
                origin = lax.rem(my - d + N_DEV, N_DEV)
                xg[:, pl.ds(origin * k_loc, k_loc)] = recv_buf[d - 1]

        origin = lax.rem(my - k + N_DEV, N_DEV)
        wb = w_ref[pl.ds(origin * k_loc, k_loc), :].astype(jnp.float8_e4m3fn)
        partial = jnp.dot(
            xg[:, pl.ds(origin * k_loc, k_loc)], wb,
            preferred_element_type=jnp.float32,
        )

        @pl.when(k == 0)
        def _():
            out_ref[:, :] = partial

        @pl.when(k != 0)
        def _():
            out_ref[:, :] += partial

        @pl.when(k == N_DEV - 1)
        def _():
            scale = sx_ref[0] * sw_ref[0]
            out_ref[:, :] = jnp.maximum(out_ref[:, :] * scale, 0.0)

    return pl.pallas_call(
        body,
        grid=(n_steps, N_DEV),
        out_shape=jax.ShapeDtypeStruct((m_loc, n), jnp.float32),
        in_specs=[
            pl.BlockSpec(memory_space=pltpu.MemorySpace.HBM),
            pl.BlockSpec((k_glob, N_TILE), lambda j, k: (0, j)),
            pl.BlockSpec(memory_space=pltpu.SMEM),
            pl.BlockSpec(memory_space=pltpu.SMEM),
        ],
        out_specs=pl.BlockSpec((m_loc, N_TILE), lambda j, k: (0, j)),
        scratch_shapes=[
            pltpu.VMEM((m_glob, k_loc), jnp.float32),
            pltpu.VMEM((m_glob, k_loc), jnp.float8_e4m3fn),
            pltpu.VMEM((m_loc, k_glob), jnp.float8_e4m3fn),
            pltpu.VMEM((N_DEV - 1, m_loc, k_loc), jnp.float8_e4m3fn),
            pltpu.SemaphoreType.DMA,
            pltpu.SemaphoreType.DMA((N_DEV - 1,)),
            pltpu.SemaphoreType.DMA((N_DEV - 1,)),
        ],
        compiler_params=pltpu.CompilerParams(
            collective_id=0,
            dimension_semantics=("arbitrary", "arbitrary"),
            vmem_limit_bytes=56 * 1024 * 1024,
        ),
    )(x, w_mat, scale_x, scale_w)


# device time: 112722 ns/iter; 1.4861x vs baseline; 1.4861x over previous
import jax
import jax.numpy as jnp
from jax import lax
from jax.experimental import pallas as pl
from jax.experimental.pallas import tpu as pltpu

N_DEV = 4
N_TILE = 512


def kernel(x, w_mat, scale_x, scale_w):
    m_glob, k_loc = x.shape
    k_glob, n = w_mat.shape
    m_loc = m_glob // N_DEV
    n_steps = n // N_TILE

    def body(x_hbm, w_ref, sx_ref, sw_ref, out_ref,
             xf32, xsend, xg, recv_buf, copy_sems, send_sems, recv_sems):
        j = pl.program_id(0)
        my = lax.axis_index("i")
        scale = sx_ref[0] * sw_ref[0]

        def mk_rdma(d):
            peer = lax.rem(my + d, N_DEV)
            return pltpu.make_async_remote_copy(
                src_ref=xsend.at[pl.ds(peer * m_loc, m_loc), :],
                dst_ref=recv_buf.at[d - 1],
                send_sem=send_sems.at[d - 1],
                recv_sem=recv_sems.at[d - 1],
                device_id=(peer,),
                device_id_type=pl.DeviceIdType.MESH,
            )

        def mk_dma(d):
            row = lax.rem(my + d, N_DEV) * m_loc
            return pltpu.make_async_copy(
                x_hbm.at[pl.ds(row, m_loc), :],
                xf32.at[pl.ds(row, m_loc), :],
                copy_sems.at[d],
            )

        @pl.when(j == 0)
        def _():
            barrier_sem = pltpu.get_barrier_semaphore()
            for d in range(1, N_DEV):
                peer = lax.rem(my + d, N_DEV)
                pl.semaphore_signal(
                    barrier_sem, inc=1,
                    device_id=(peer,), device_id_type=pl.DeviceIdType.MESH,
                )

            dmas = [mk_dma(d) for d in range(N_DEV)]
            for dma in dmas:
                dma.start()

            pl.semaphore_wait(barrier_sem, N_DEV - 1)

            for d in range(1, N_DEV):
                dmas[d].wait()
                row = lax.rem(my + d, N_DEV) * m_loc
                xsend[pl.ds(row, m_loc), :] = (
                    xf32[pl.ds(row, m_loc), :].astype(jnp.float8_e4m3fn)
                )
                mk_rdma(d).start()

            dmas[0].wait()
            xg[:, pl.ds(my * k_loc, k_loc)] = (
                xf32[pl.ds(my * m_loc, m_loc), :].astype(jnp.float8_e4m3fn)
            )
            wb = w_ref[pl.ds(my * k_loc, k_loc), :].astype(jnp.float8_e4m3fn)
            out_ref[:, :] = jnp.dot(
                xg[:, pl.ds(my * k_loc, k_loc)], wb,
                preferred_element_type=jnp.float32,
            )

            for d in range(1, N_DEV):
                rdma = mk_rdma(d)
                rdma.wait_send()
                rdma.wait_recv()
                origin = lax.rem(my - d + N_DEV, N_DEV)
                xg[:, pl.ds(origin * k_loc, k_loc)] = recv_buf[d - 1]
                wb = w_ref[pl.ds(origin * k_loc, k_loc), :].astype(
                    jnp.float8_e4m3fn
                )
                out_ref[:, :] += jnp.dot(
                    xg[:, pl.ds(origin * k_loc, k_loc)], wb,
                    preferred_element_type=jnp.float32,
                )

            out_ref[:, :] = jnp.maximum(out_ref[:, :] * scale, 0.0)

        @pl.when(j != 0)
        def _():
            wb = w_ref[:, :].astype(jnp.float8_e4m3fn)
            acc = jnp.dot(xg[:, :], wb, preferred_element_type=jnp.float32)
            out_ref[:, :] = jnp.maximum(acc * scale, 0.0)

    return pl.pallas_call(
        body,
        grid=(n_steps,),
        out_shape=jax.ShapeDtypeStruct((m_loc, n), jnp.float32),
        in_specs=[
            pl.BlockSpec(memory_space=pltpu.MemorySpace.HBM),
            pl.BlockSpec((k_glob, N_TILE), lambda j: (0, j)),
            pl.BlockSpec(memory_space=pltpu.SMEM),
            pl.BlockSpec(memory_space=pltpu.SMEM),
        ],
        out_specs=pl.BlockSpec((m_loc, N_TILE), lambda j: (0, j)),
        scratch_shapes=[
            pltpu.VMEM((m_glob, k_loc), jnp.float32),
            pltpu.VMEM((m_glob, k_loc), jnp.float8_e4m3fn),
            pltpu.VMEM((m_loc, k_glob), jnp.float8_e4m3fn),
            pltpu.VMEM((N_DEV - 1, m_loc, k_loc), jnp.float8_e4m3fn),
            pltpu.SemaphoreType.DMA((N_DEV,)),
            pltpu.SemaphoreType.DMA((N_DEV - 1,)),
            pltpu.SemaphoreType.DMA((N_DEV - 1,)),
        ],
        compiler_params=pltpu.CompilerParams(
            collective_id=0,
            dimension_semantics=("arbitrary",),
            vmem_limit_bytes=56 * 1024 * 1024,
        ),
    )(x, w_mat, scale_x, scale_w)
